# baseline (speedup 1.0000x reference)
import functools

import jax
import jax.numpy as jnp
from jax import lax
from jax.experimental import pallas as pl
from jax.experimental.pallas import tpu as pltpu
from jax.experimental.pallas import tpu_sc as plsc

_L = 14
_P = _L * _L
_R = 5
_THRESHOLD = 0.6


def _nms_body(bpw, h_hbm, out1_hbm, out2_hbm, heat_v, out1_v, out2_v):
    info = plsc.get_sparse_core_info()
    nc, lanes_n = info.num_cores, info.num_lanes
    chunk = bpw * _P
    ngroups = bpw // lanes_n

    wid = lax.axis_index("s") * nc + lax.axis_index("c")
    base = wid * chunk
    pltpu.sync_copy(h_hbm.at[pl.ds(base, chunk)], heat_v)

    lanes = lax.iota(jnp.int32, lanes_n)
    zeros_f = jnp.zeros((lanes_n,), jnp.float32)
    ones_f = jnp.full((lanes_n,), 1.0, jnp.float32)

    def group_body(g, carry):
        bvec = (g * lanes_n + lanes) * _P

        ims = []
        for r in range(4):

            def scan_rows(i, c):
                cm, ci = c
                row = bvec + i * _L
                for j in range(_L):
                    idx = row + j
                    v = plsc.load_gather(heat_v, [idx])
                    cond = v > cm
                    cm = jnp.where(cond, v, cm)
                    ci = jnp.where(cond, idx, ci)
                return cm, ci

            init = (jnp.full((lanes_n,), _THRESHOLD, jnp.float32), bvec)
            _, ci = lax.fori_loop(0, _L, scan_rows, init)
            im = ci - bvec
            ims.append(im)

            if r < 3:
                x = im // _L
                y = im - x * _L

                def sup_body(t, ci):
                    dx = t - _R
                    xn = x + dx
                    okx = (xn >= 0) & (xn < _L)
                    for dyj in range(2 * _R):
                        dy = dyj - _R
                        yn = y + dy
                        ok = okx & (yn >= 0) & (yn < _L)
                        tgt = ci + (dx * _L + dy)
                        plsc.store_scatter(heat_v, [tgt], zeros_f, mask=ok)
                    return ci

                lax.fori_loop(0, 2 * _R, sup_body, ci)

        xs = [im // _L for im in ims]
        ys = [im - (im // _L) * _L for im in ims]
        pairs = [(0, 1), (0, 2), (0, 3), (1, 2), (1, 3), (2, 3)]
        best = jnp.full((lanes_n,), -1, jnp.int32)
        c1x, c1y, c2x, c2y = xs[0], ys[0], xs[1], ys[1]
        for a, b in pairs:
            dxx = xs[b] - xs[a]
            dyy = ys[b] - ys[a]
            d = dxx * dxx + dyy * dyy
            cond = d > best
            best = jnp.where(cond, d, best)
            c1x = jnp.where(cond, xs[a], c1x)
            c1y = jnp.where(cond, ys[a], c1y)
            c2x = jnp.where(cond, xs[b], c2x)
            c2y = jnp.where(cond, ys[b], c2y)

        ax = 2 * (c2x - c1x)
        ay = 2 * (c2y - c1y)
        kk = c2x * c2x + c2y * c2y - c1x * c1x - c1y * c1y

        def vor_rows(i, c):
            rbase = i * ax - kk
            row = bvec + i * _L
            for j in range(_L):
                lhs = rbase + j * ay
                m = lhs < 0
                m1 = jnp.where(m, ones_f, zeros_f)
                m2 = jnp.where(m, zeros_f, ones_f)
                tgt = row + j
                plsc.store_scatter(out1_v, [tgt], m1)
                plsc.store_scatter(out2_v, [tgt], m2)
            return c

        lax.fori_loop(0, _L, vor_rows, 0)
        return carry

    lax.fori_loop(0, ngroups, group_body, 0)

    pltpu.sync_copy(out1_v, out1_hbm.at[pl.ds(base, chunk)])
    pltpu.sync_copy(out2_v, out2_hbm.at[pl.ds(base, chunk)])


@functools.partial(jax.jit, static_argnums=(1,))
def _nms_run(hflat, bpw):
    chunk = bpw * _P
    n = hflat.shape[0]
    mesh = plsc.VectorSubcoreMesh(core_axis_name="c", subcore_axis_name="s")
    out = pl.kernel(
        functools.partial(_nms_body, bpw),
        out_type=(
            jax.ShapeDtypeStruct((n,), jnp.float32),
            jax.ShapeDtypeStruct((n,), jnp.float32),
        ),
        mesh=mesh,
        compiler_params=pltpu.CompilerParams(needs_layout_passes=False),
        scratch_types=[
            pltpu.VMEM((chunk,), jnp.float32),
            pltpu.VMEM((chunk,), jnp.float32),
            pltpu.VMEM((chunk,), jnp.float32),
        ],
    )(hflat)
    return out


def kernel(heatmap):
    b = heatmap.shape[0]
    info = plsc.get_sparse_core_info()
    nw = info.num_cores * info.num_subcores
    bpw = b // nw
    hflat = heatmap.reshape(b * _P)
    o1, o2 = _nms_run(hflat, bpw)
    return (o1.reshape(b, 1, _L, _L), o2.reshape(b, 1, _L, _L))
